# baseline (speedup 1.0000x reference)
import functools

import jax
import jax.numpy as jnp
from jax import lax
from jax.experimental import pallas as pl
from jax.experimental.pallas import tpu as pltpu
from jax.experimental.pallas import tpu_sc as plsc

B, N, K, D, DOUT = 4, 8192, 16, 64, 64
E = B * N * K
BN_EPS = 1e-6
LEAKY_SLOPE = 0.2
CPAD = 8
CH = 128
EBLK = 2048


def _sc_gather(feats_flat, coords_pad, gidx):
    info = plsc.get_sparse_core_info()
    nwork = info.num_cores * info.num_subcores
    epw = E // nwork
    nch = epw // CH
    idx3 = gidx.reshape(nwork, nch, CH)
    mesh = plsc.VectorSubcoreMesh(core_axis_name="c", subcore_axis_name="s")

    @functools.partial(
        pl.kernel,
        mesh=mesh,
        compiler_params=pltpu.CompilerParams(use_tc_tiling_on_sc=False),
        out_type=[
            jax.ShapeDtypeStruct((E, 2, D), jnp.float32),
            jax.ShapeDtypeStruct((E, CPAD), jnp.float32),
        ],
        scratch_types=[
            pltpu.VMEM((nch, CH), jnp.int32),
            pltpu.VMEM((2, CH, D), jnp.float32),
            pltpu.VMEM((2, CH, CPAD), jnp.float32),
            pltpu.SemaphoreType.DMA,
            pltpu.SemaphoreType.DMA,
            pltpu.SemaphoreType.DMA,
            pltpu.SemaphoreType.DMA,
        ],
    )
    def k(feats_hbm, coords_hbm, idx_hbm, out_hbm, ncrd_hbm,
          idx_v, rows_v, crd_v, sf0, sf1, sc0, sc1):
        wid = lax.axis_index("s") * info.num_cores + lax.axis_index("c")
        wbase = wid * epw
        pltpu.sync_copy(idx_hbm.at[wid], idx_v)
        sf = (sf0, sf1)
        sc = (sc0, sc1)

        def issue(c, b):
            cp_f = pltpu.async_copy(feats_hbm.at[idx_v.at[c]], rows_v.at[b], sf[b])
            cp_c = pltpu.async_copy(coords_hbm.at[idx_v.at[c]], crd_v.at[b], sc[b])
            return cp_f, cp_c

        def drain_and_write(c, b):
            pltpu.make_async_copy(feats_hbm.at[idx_v.at[0]], rows_v.at[b], sf[b]).wait()
            pltpu.make_async_copy(coords_hbm.at[idx_v.at[0]], crd_v.at[b], sc[b]).wait()
            off = wbase + c * CH
            pltpu.sync_copy(rows_v.at[b], out_hbm.at[pl.ds(off, CH), 0])
            pltpu.sync_copy(crd_v.at[b], ncrd_hbm.at[pl.ds(off, CH)])

        issue(0, 0)
        issue(1, 1)

        def body(i, carry):
            c0 = i * 2
            for b in range(2):
                drain_and_write(c0 + b, b)
                issue(c0 + b + 2, b)
            return carry

        lax.fori_loop(0, (nch - 2) // 2, body, 0)
        drain_and_write(nch - 2, 0)
        drain_and_write(nch - 1, 1)

    return k(feats_flat, coords_pad, idx3)


NGRID = E // EBLK


def _tc_body(o_in_ref, nc_ref, cr_ref, wa_ref, wc_ref, b_ref,
             o_ref, rel_ref, ybuf, sems):
    del o_in_ref
    i = pl.program_id(0)
    ncr = nc_ref[...]
    cr = cr_ref[...]
    ec = jnp.broadcast_to(cr[:, None, :],
                          (EBLK // K, K, CPAD)).reshape(EBLK, CPAD)
    rel = ec - ncr
    d2 = jnp.sum(rel * rel, axis=1, keepdims=True) + 1e-12
    dist = jnp.sqrt(d2)
    lane = lax.broadcasted_iota(jnp.int32, (1, CPAD), 1)
    onehot3 = (lane == 3).astype(jnp.float32)
    ecd = ec + (jnp.broadcast_to(dist, (EBLK, CPAD))
                * jnp.broadcast_to(onehot3, (EBLK, CPAD)))
    x = (jnp.dot(ecd, wa_ref[...], preferred_element_type=jnp.float32)
         + jnp.dot(ncr, wc_ref[...], preferred_element_type=jnp.float32)
         + jnp.broadcast_to(b_ref[...], (EBLK, DOUT)))
    y = jnp.maximum(x, LEAKY_SLOPE * x)
    rel_ref[...] = y

    slot = lax.rem(i, 2)

    def _cp(j, s):
        return pltpu.make_async_copy(
            ybuf.at[s],
            o_ref.at[pl.ds(j * EBLK, EBLK), 1],
            sems.at[s])

    @pl.when(i >= 2)
    def _():
        _cp(i - 2, slot).wait()

    @pl.when(slot == 0)
    def _():
        ybuf[0] = y

    @pl.when(slot == 1)
    def _():
        ybuf[1] = y

    _cp(i, slot).start()

    @pl.when(i == NGRID - 1)
    def _():
        _cp(i - 1, 1 - slot).wait()
        _cp(i, slot).wait()


def _tc_dense(out1, nbr_coords, coords_flat, wa, wc, beff):
    return pl.pallas_call(
        _tc_body,
        grid=(NGRID,),
        in_specs=[
            pl.BlockSpec(memory_space=pl.ANY),
            pl.BlockSpec((EBLK, CPAD), lambda i: (i, 0)),
            pl.BlockSpec((EBLK // K, CPAD), lambda i: (i, 0)),
            pl.BlockSpec((CPAD, DOUT), lambda i: (0, 0)),
            pl.BlockSpec((CPAD, DOUT), lambda i: (0, 0)),
            pl.BlockSpec((1, DOUT), lambda i: (0, 0)),
        ],
        out_specs=[
            pl.BlockSpec(memory_space=pl.ANY),
            pl.BlockSpec((EBLK, DOUT), lambda i: (i, 0)),
        ],
        out_shape=[
            jax.ShapeDtypeStruct((E, 2, D), jnp.float32),
            jax.ShapeDtypeStruct((E, DOUT), jnp.float32),
        ],
        scratch_shapes=[
            pltpu.VMEM((2, EBLK, DOUT), jnp.float32),
            pltpu.SemaphoreType.DMA((2,)),
        ],
        input_output_aliases={0: 0},
    )(out1, nbr_coords, coords_flat, wa, wc, beff)


def kernel(coords, features, neighbor_indices, W, b, gamma, beta, training):
    del training
    feats_flat = features.reshape(B * N, D)
    cpad = jnp.pad(coords, ((0, 0), (0, 0), (0, CPAD - 3)))
    coords_flat = cpad.reshape(B * N, CPAD)
    gidx = (neighbor_indices
            + (jnp.arange(B, dtype=jnp.int32) * N)[:, None, None]).reshape(E)

    scale = gamma / jnp.sqrt(1.0 + BN_EPS)
    w_eff = W * scale[None, :]
    beff = (b * scale + beta).reshape(1, DOUT)
    wa = jnp.concatenate([w_eff[4:7] + w_eff[1:4], w_eff[0:1],
                          jnp.zeros((CPAD - 4, DOUT), jnp.float32)], axis=0)
    wc = jnp.concatenate([w_eff[7:10] - w_eff[1:4],
                          jnp.zeros((CPAD - 3, DOUT), jnp.float32)], axis=0)

    out1, nbr_coords = _sc_gather(feats_flat, coords_flat, gidx)

    out_flat, rel_flat = _tc_dense(out1, nbr_coords, coords_flat,
                                   wa, wc, beff)
    return (out_flat.reshape(B, N, K, D + DOUT),
            rel_flat.reshape(B, N, K, DOUT))


# speedup vs baseline: 2.7523x; 2.7523x over previous
import functools

import jax
import jax.numpy as jnp
from jax import lax
from jax.experimental import pallas as pl
from jax.experimental.pallas import tpu as pltpu
from jax.experimental.pallas import tpu_sc as plsc

B, N, K, D, DOUT = 4, 8192, 16, 64, 64
E = B * N * K
BN_EPS = 1e-6
LEAKY_SLOPE = 0.2
CPAD = 8
CH = 128
CH2 = 256
EBLK = 2048
NGRID = E // EBLK


def _sc_gather(feats_flat, coords_pad, gidx):
    info = plsc.get_sparse_core_info()
    nwork = info.num_cores * info.num_subcores
    epw = E // nwork
    nch = epw // CH
    idx3 = gidx.reshape(nwork, nch, CH)
    mesh = plsc.VectorSubcoreMesh(core_axis_name="c", subcore_axis_name="s")

    @functools.partial(
        pl.kernel,
        mesh=mesh,
        compiler_params=pltpu.CompilerParams(use_tc_tiling_on_sc=False),
        out_type=[
            jax.ShapeDtypeStruct((E, D), jnp.float32),
            jax.ShapeDtypeStruct((E, CPAD), jnp.float32),
        ],
        scratch_types=[
            pltpu.VMEM((nch, CH), jnp.int32),
            pltpu.VMEM((2, CH, D), jnp.float32),
            pltpu.VMEM((2, CH, CPAD), jnp.float32),
            pltpu.SemaphoreType.DMA,
            pltpu.SemaphoreType.DMA,
            pltpu.SemaphoreType.DMA,
            pltpu.SemaphoreType.DMA,
        ],
    )
    def k(feats_hbm, coords_hbm, idx_hbm, nf_hbm, ncrd_hbm,
          idx_v, rows_v, crd_v, sf0, sf1, sc0, sc1):
        wid = lax.axis_index("s") * info.num_cores + lax.axis_index("c")
        wbase = wid * epw
        pltpu.sync_copy(idx_hbm.at[wid], idx_v)
        sf = (sf0, sf1)
        sc = (sc0, sc1)

        def issue(c, b):
            pltpu.async_copy(feats_hbm.at[idx_v.at[c]], rows_v.at[b], sf[b])
            pltpu.async_copy(coords_hbm.at[idx_v.at[c]], crd_v.at[b], sc[b])

        def drain_and_write(c, b):
            pltpu.make_async_copy(feats_hbm.at[idx_v.at[0]], rows_v.at[b], sf[b]).wait()
            pltpu.make_async_copy(coords_hbm.at[idx_v.at[0]], crd_v.at[b], sc[b]).wait()
            off = wbase + c * CH
            pltpu.sync_copy(rows_v.at[b], nf_hbm.at[pl.ds(off, CH)])
            pltpu.sync_copy(crd_v.at[b], ncrd_hbm.at[pl.ds(off, CH)])

        issue(0, 0)
        issue(1, 1)

        def body(i, carry):
            c0 = i * 2
            for b in range(2):
                drain_and_write(c0 + b, b)
                issue(c0 + b + 2, b)
            return carry

        lax.fori_loop(0, (nch - 2) // 2, body, 0)
        drain_and_write(nch - 2, 0)
        drain_and_write(nch - 1, 1)

    return k(feats_flat, coords_pad, idx3)


def _sc_assemble(nf, rel):
    info = plsc.get_sparse_core_info()
    nwork = info.num_cores * info.num_subcores
    epw = E // nwork
    nch = epw // CH2
    mesh = plsc.VectorSubcoreMesh(core_axis_name="c", subcore_axis_name="s")

    @functools.partial(
        pl.kernel,
        mesh=mesh,
        compiler_params=pltpu.CompilerParams(use_tc_tiling_on_sc=False),
        out_type=jax.ShapeDtypeStruct((E, D + DOUT), jnp.float32),
        scratch_types=[
            pltpu.VMEM((2, CH2, D), jnp.float32),
            pltpu.VMEM((2, CH2, DOUT), jnp.float32),
            pltpu.SemaphoreType.DMA,
            pltpu.SemaphoreType.DMA,
        ],
    )
    def k(nf_hbm, rel_hbm, out_hbm, nf_v, rel_v, s0, s1):
        wid = lax.axis_index("s") * info.num_cores + lax.axis_index("c")
        wbase = wid * epw
        ss = (s0, s1)

        def issue(c, b):
            off = wbase + c * CH2
            pltpu.async_copy(nf_hbm.at[pl.ds(off, CH2)], nf_v.at[b], ss[b])
            pltpu.async_copy(rel_hbm.at[pl.ds(off, CH2)], rel_v.at[b], ss[b])

        def drain_and_write(c, b):
            off = wbase + c * CH2
            pltpu.make_async_copy(nf_hbm.at[pl.ds(off, CH2)], nf_v.at[b], ss[b]).wait()
            pltpu.make_async_copy(rel_hbm.at[pl.ds(off, CH2)], rel_v.at[b], ss[b]).wait()
            pltpu.sync_copy(nf_v.at[b], out_hbm.at[pl.ds(off, CH2), pl.ds(0, D)])
            pltpu.sync_copy(rel_v.at[b], out_hbm.at[pl.ds(off, CH2), pl.ds(D, DOUT)])

        issue(0, 0)
        issue(1, 1)

        def body(i, carry):
            c0 = i * 2
            for b in range(2):
                drain_and_write(c0 + b, b)
                issue(c0 + b + 2, b)
            return carry

        lax.fori_loop(0, (nch - 2) // 2, body, 0)
        drain_and_write(nch - 2, 0)
        drain_and_write(nch - 1, 1)

    return k(nf, rel)


def _tc_body(nc_ref, cr_ref, wa_ref, wc_ref, b_ref, rel_ref):
    ncr = nc_ref[...]
    cr = cr_ref[...]
    ec = jnp.broadcast_to(cr[:, None, :],
                          (EBLK // K, K, CPAD)).reshape(EBLK, CPAD)
    rel = ec - ncr
    d2 = jnp.sum(rel * rel, axis=1, keepdims=True) + 1e-12
    dist = jnp.sqrt(d2)
    lane = lax.broadcasted_iota(jnp.int32, (1, CPAD), 1)
    onehot3 = (lane == 3).astype(jnp.float32)
    ecd = ec + (jnp.broadcast_to(dist, (EBLK, CPAD))
                * jnp.broadcast_to(onehot3, (EBLK, CPAD)))
    x = (jnp.dot(ecd, wa_ref[...], preferred_element_type=jnp.float32)
         + jnp.dot(ncr, wc_ref[...], preferred_element_type=jnp.float32)
         + jnp.broadcast_to(b_ref[...], (EBLK, DOUT)))
    rel_ref[...] = jnp.maximum(x, LEAKY_SLOPE * x)


def _tc_dense(nbr_coords, coords_flat, wa, wc, beff):
    return pl.pallas_call(
        _tc_body,
        grid=(NGRID,),
        in_specs=[
            pl.BlockSpec((EBLK, CPAD), lambda i: (i, 0)),
            pl.BlockSpec((EBLK // K, CPAD), lambda i: (i, 0)),
            pl.BlockSpec((CPAD, DOUT), lambda i: (0, 0)),
            pl.BlockSpec((CPAD, DOUT), lambda i: (0, 0)),
            pl.BlockSpec((1, DOUT), lambda i: (0, 0)),
        ],
        out_specs=pl.BlockSpec((EBLK, DOUT), lambda i: (i, 0)),
        out_shape=jax.ShapeDtypeStruct((E, DOUT), jnp.float32),
    )(nbr_coords, coords_flat, wa, wc, beff)


def kernel(coords, features, neighbor_indices, W, b, gamma, beta, training):
    del training
    feats_flat = features.reshape(B * N, D)
    cpad = jnp.pad(coords, ((0, 0), (0, 0), (0, CPAD - 3)))
    coords_flat = cpad.reshape(B * N, CPAD)
    gidx = (neighbor_indices
            + (jnp.arange(B, dtype=jnp.int32) * N)[:, None, None]).reshape(E)

    scale = gamma / jnp.sqrt(1.0 + BN_EPS)
    w_eff = W * scale[None, :]
    beff = (b * scale + beta).reshape(1, DOUT)
    wa = jnp.concatenate([w_eff[4:7] + w_eff[1:4], w_eff[0:1],
                          jnp.zeros((CPAD - 4, DOUT), jnp.float32)], axis=0)
    wc = jnp.concatenate([w_eff[7:10] - w_eff[1:4],
                          jnp.zeros((CPAD - 3, DOUT), jnp.float32)], axis=0)

    nf, nbr_coords = _sc_gather(feats_flat, coords_flat, gidx)
    rel_flat = _tc_dense(nbr_coords, coords_flat, wa, wc, beff)
    out_flat = _sc_assemble(nf, rel_flat)
    return (out_flat.reshape(B, N, K, D + DOUT),
            rel_flat.reshape(B, N, K, DOUT))


# speedup vs baseline: 4.2216x; 1.5338x over previous
import functools

import jax
import jax.numpy as jnp
from jax import lax
from jax.experimental import pallas as pl
from jax.experimental.pallas import tpu as pltpu
from jax.experimental.pallas import tpu_sc as plsc

B, N, K, D, DOUT = 4, 8192, 16, 64, 64
E = B * N * K
BN_EPS = 1e-6
LEAKY_SLOPE = 0.2
CPAD = 8
CH = 128
GPC = CH // K
NSLOT = 4
NJ = DOUT // 16


def _sc_fused(feats_flat, coords_pad, coords_1d, gidx, wrow, beff):
    info = plsc.get_sparse_core_info()
    nwork = info.num_cores * info.num_subcores
    epw = E // nwork
    nch = epw // CH
    npt = epw // K
    idx3 = gidx.reshape(nwork, nch, CH)
    mesh = plsc.VectorSubcoreMesh(core_axis_name="c", subcore_axis_name="s")

    @functools.partial(
        pl.kernel,
        mesh=mesh,
        compiler_params=pltpu.CompilerParams(use_tc_tiling_on_sc=False,
                                             needs_layout_passes=False),
        out_type=[
            jax.ShapeDtypeStruct((E, D + DOUT), jnp.float32),
            jax.ShapeDtypeStruct((E, DOUT), jnp.float32),
        ],
        scratch_types=(
            [pltpu.VMEM((nch, CH), jnp.int32),
             pltpu.VMEM((npt * CPAD + 16,), jnp.float32),
             pltpu.VMEM((NSLOT, CH, D), jnp.float32),
             pltpu.VMEM((NSLOT, CH, CPAD), jnp.float32),
             pltpu.VMEM((NSLOT, CH, DOUT), jnp.float32),
             pltpu.VMEM((8, DOUT), jnp.float32),
             pltpu.VMEM((1, DOUT), jnp.float32)]
            + [pltpu.SemaphoreType.DMA] * (4 * NSLOT)
        ),
    )
    def k(feats_hbm, coords_hbm, c1d_hbm, idx_hbm, w_hbm, b_hbm, out_hbm, rel_hbm,
          idx_v, cpt_v, rows_v, crd_v, rel_v, w_v, b_v, *sems):
        sf = sems[0:NSLOT]
        sc = sems[NSLOT:2 * NSLOT]
        wf = sems[2 * NSLOT:3 * NSLOT]
        wr = sems[3 * NSLOT:4 * NSLOT]
        wid = lax.axis_index("s") * info.num_cores + lax.axis_index("c")
        wbase = wid * epw
        pltpu.sync_copy(idx_hbm.at[wid], idx_v)
        pltpu.sync_copy(c1d_hbm.at[pl.ds(wid * npt * CPAD, npt * CPAD)],
                        cpt_v.at[pl.ds(0, npt * CPAD)])
        pltpu.sync_copy(w_hbm, w_v)
        pltpu.sync_copy(b_hbm, b_v)

        iota = lax.iota(jnp.int32, 16)
        wvec = [[w_v[c, pl.ds(16 * j, 16)] for j in range(NJ)] for c in range(8)]
        bvec = [b_v[0, pl.ds(16 * j, 16)] for j in range(NJ)]

        def issue(c, q):
            pltpu.async_copy(feats_hbm.at[idx_v.at[c]], rows_v.at[q], sf[q])
            pltpu.async_copy(coords_hbm.at[idx_v.at[c]], crd_v.at[q], sc[q])

        def wait_gather(c, q):
            pltpu.make_async_copy(feats_hbm.at[idx_v.at[0]], rows_v.at[q], sf[q]).wait()
            pltpu.make_async_copy(coords_hbm.at[idx_v.at[0]], crd_v.at[q], sc[q]).wait()

        def compute(c, q):
            crd2 = crd_v.at[q]
            rel2 = rel_v.at[q]

            def group(g, carry):
                p = c * GPC + g
                ev = cpt_v[pl.ds(p * CPAD, 16)]
                ecx = ev[0]
                ecy = ev[1]
                ecz = ev[2]
                bias = [bvec[j] + ecx * wvec[0][j] + ecy * wvec[1][j]
                        + ecz * wvec[2][j] for j in range(NJ)]
                ridx = g * 16 + iota
                nx = plsc.load_gather(crd2, [ridx, jnp.full((16,), 0, jnp.int32)])
                ny = plsc.load_gather(crd2, [ridx, jnp.full((16,), 1, jnp.int32)])
                nz = plsc.load_gather(crd2, [ridx, jnp.full((16,), 2, jnp.int32)])
                dx = ecx - nx
                dy = ecy - ny
                dz = ecz - nz
                d2 = dx * dx + dy * dy + dz * dz + 1e-12
                bits = lax.bitcast_convert_type(d2, jnp.int32)
                y0 = lax.bitcast_convert_type(
                    0x5F3759DF - lax.shift_right_logical(bits, 1), jnp.float32)
                y1 = y0 * (1.5 - 0.5 * d2 * y0 * y0)
                y2 = y1 * (1.5 - 0.5 * d2 * y1 * y1)
                y3 = y2 * (1.5 - 0.5 * d2 * y2 * y2)
                dist = d2 * y3
                for l in range(16):
                    row = g * 16 + l
                    ds_ = dist[l]
                    nxs = nx[l]
                    nys = ny[l]
                    nzs = nz[l]
                    for j in range(NJ):
                        x = (bias[j] + ds_ * wvec[3][j] + nxs * wvec[4][j]
                             + nys * wvec[5][j] + nzs * wvec[6][j])
                        yv = jnp.maximum(x, LEAKY_SLOPE * x)
                        rel2[row, pl.ds(16 * j, 16)] = yv
                return carry

            lax.fori_loop(0, GPC, group, 0)

        def write(c, q):
            off = wbase + c * CH
            pltpu.async_copy(rows_v.at[q], out_hbm.at[pl.ds(off, CH), pl.ds(0, D)], wf[q])
            pltpu.async_copy(rel_v.at[q], out_hbm.at[pl.ds(off, CH), pl.ds(D, DOUT)], wr[q])
            pltpu.async_copy(rel_v.at[q], rel_hbm.at[pl.ds(off, CH)], wr[q])

        def wait_write(c, q):
            off = wbase + c * CH
            pltpu.make_async_copy(rel_v.at[q], out_hbm.at[pl.ds(off, CH), pl.ds(D, DOUT)], wr[q]).wait()
            pltpu.make_async_copy(rel_v.at[q], rel_hbm.at[pl.ds(off, CH)], wr[q]).wait()

        def wait_nf(c, q):
            off = wbase + c * CH
            pltpu.make_async_copy(rows_v.at[q], out_hbm.at[pl.ds(off, CH), pl.ds(0, D)], wf[q]).wait()

        issue(0, 0)
        issue(1, 1)
        for c in range(4):
            q = c % NSLOT
            wait_gather(c, q)
            compute(c, q)
            write(c, q)
            if c >= 2:
                wait_nf(c - 2, (c + 2) % NSLOT)
            issue(c + 2, (c + 2) % NSLOT)

        def body(i, carry):
            c0 = i * NSLOT
            for b in range(NSLOT):
                c = c0 + b
                q = b
                q2 = (b + 2) % NSLOT
                wait_gather(c, q)
                wait_write(c - NSLOT, q)
                compute(c, q)
                write(c, q)
                wait_nf(c - 2, q2)
                issue(c + 2, q2)
            return carry

        lax.fori_loop(1, nch // NSLOT - 1, body, 0)

        for cc in range(nch - NSLOT, nch):
            q = cc % NSLOT
            wait_gather(cc, q)
            wait_write(cc - NSLOT, q)
            compute(cc, q)
            write(cc, q)
            if cc < nch - 2:
                wait_nf(cc - 2, (cc + 2) % NSLOT)
                issue(cc + 2, (cc + 2) % NSLOT)
        for cc in range(nch - NSLOT, nch):
            q = cc % NSLOT
            wait_write(cc, q)
        wait_nf(nch - 2, (nch - 2) % NSLOT)
        wait_nf(nch - 1, (nch - 1) % NSLOT)
        wait_nf(nch - 4, (nch - 4) % NSLOT)
        wait_nf(nch - 3, (nch - 3) % NSLOT)

    return k(feats_flat, coords_pad, coords_1d, idx3, wrow, beff)


def kernel(coords, features, neighbor_indices, W, b, gamma, beta, training):
    del training
    feats_flat = features.reshape(B * N, D)
    cpad = jnp.pad(coords, ((0, 0), (0, 0), (0, CPAD - 3)))
    coords_flat = cpad.reshape(B * N, CPAD)
    gidx = (neighbor_indices
            + (jnp.arange(B, dtype=jnp.int32) * N)[:, None, None]).reshape(E)

    scale = gamma / jnp.sqrt(1.0 + BN_EPS)
    w_eff = W * scale[None, :]
    beff = (b * scale + beta).reshape(1, DOUT)
    wrow = jnp.concatenate([w_eff[4:7] + w_eff[1:4], w_eff[0:1],
                            w_eff[7:10] - w_eff[1:4],
                            jnp.zeros((1, DOUT), jnp.float32)], axis=0)

    out_flat, rel_flat = _sc_fused(feats_flat, coords_flat,
                                   coords_flat.reshape(-1), gidx, wrow, beff)
    return (out_flat.reshape(B, N, K, D + DOUT),
            rel_flat.reshape(B, N, K, DOUT))
